# Initial kernel scaffold; baseline (speedup 1.0000x reference)
import functools

import jax
import jax.numpy as jnp
from jax import lax
from jax.experimental import pallas as pl
from jax.experimental.pallas import tpu as pltpu
from jax.experimental.pallas import tpu_sc as plsc

N_NODES = 10000
N_EDGES = 320000
D_FEAT = 128

NC = 2
NS = 16
NW = NC * NS
E_TILE = N_EDGES // NW
CHUNK = 80
CHUNKS = E_TILE // CHUNK
SUB = CHUNK // 16
ROWS_TILE = N_NODES // NS

_mesh = plsc.VectorSubcoreMesh(
    core_axis_name="c", subcore_axis_name="s", num_cores=NC, num_subcores=NS
)


@functools.partial(
    pl.kernel,
    out_type=jax.ShapeDtypeStruct((NC, N_NODES, D_FEAT), jnp.float32),
    mesh=_mesh,
    scratch_types=[
        pltpu.VMEM((CHUNKS, CHUNK), jnp.int32),
        pltpu.VMEM((CHUNKS, CHUNK), jnp.int32),
        pltpu.VMEM((E_TILE,), jnp.float32),
        pltpu.VMEM((CHUNK, D_FEAT), jnp.float32),
        pltpu.VMEM((ROWS_TILE, D_FEAT), jnp.float32),
        pltpu.VMEM_SHARED((N_NODES, D_FEAT), jnp.float32),
        pltpu.SemaphoreType.DMA,
    ],
)
def _sc_scatter(x_hbm, src_hbm, tgt_hbm, val_hbm, out_hbm,
                src_v, tgt_v, val_v, rows_v, slab_v, acc_sh, sem):
    cid = lax.axis_index("c")
    sid = lax.axis_index("s")
    wid = sid * NC + cid

    pltpu.sync_copy(src_hbm.at[wid], src_v)
    pltpu.sync_copy(tgt_hbm.at[wid], tgt_v)
    pltpu.sync_copy(val_hbm.at[wid], val_v)

    zero = jnp.zeros((16,), jnp.float32)

    def _zero_row(r, carry):
        for k in range(D_FEAT // 16):
            slab_v[r, pl.ds(k * 16, 16)] = zero
        return carry

    lax.fori_loop(0, ROWS_TILE, _zero_row, 0)
    pltpu.sync_copy(slab_v, acc_sh.at[pl.ds(sid * ROWS_TILE, ROWS_TILE)])
    plsc.subcore_barrier()

    def _chunk(j, carry):
        pltpu.async_copy(x_hbm.at[src_v.at[j]], rows_v, sem).wait()

        def _group(s, c2):
            off = pl.multiple_of(j * CHUNK + s * 16, 16)
            val16 = val_v[pl.ds(off, 16)]
            for e16 in range(16):
                sv = jnp.take(
                    val16,
                    jnp.full((16,), e16, jnp.int32),
                    mode="promise_in_bounds",
                )
                e = s * 16 + e16
                for k in range(D_FEAT // 16):
                    rows_v[e, pl.ds(k * 16, 16)] = (
                        rows_v[e, pl.ds(k * 16, 16)] * sv
                    )
            return c2

        lax.fori_loop(0, SUB, _group, 0)

        pltpu.sync_copy(rows_v, acc_sh.at[tgt_v.at[j]], add=True)
        return carry

    lax.fori_loop(0, CHUNKS, _chunk, 0)
    plsc.subcore_barrier()

    pltpu.sync_copy(acc_sh.at[pl.ds(sid * ROWS_TILE, ROWS_TILE)], slab_v)
    pltpu.sync_copy(slab_v, out_hbm.at[cid, pl.ds(sid * ROWS_TILE, ROWS_TILE)])


def _combine_body(a_ref, b_ref, o_ref):
    o_ref[...] = a_ref[...] + b_ref[...]


_combine = pl.pallas_call(
    _combine_body,
    out_shape=jax.ShapeDtypeStruct((N_NODES, D_FEAT), jnp.float32),
    grid=(8,),
    in_specs=[
        pl.BlockSpec((1250, D_FEAT), lambda i: (i, 0)),
        pl.BlockSpec((1250, D_FEAT), lambda i: (i, 0)),
    ],
    out_specs=pl.BlockSpec((1250, D_FEAT), lambda i: (i, 0)),
)


@jax.jit
def kernel(x_source, edge_index, values):
    src = edge_index[1].reshape(NW, CHUNKS, CHUNK)
    tgt = edge_index[0].reshape(NW, CHUNKS, CHUNK)
    val = values.reshape(NW, E_TILE)
    partial = _sc_scatter(x_source, src, tgt, val)
    return _combine(partial[0], partial[1])


# speedup vs baseline: 9.8814x; 9.8814x over previous
import functools

import jax
import jax.numpy as jnp
from jax import lax
from jax.experimental import pallas as pl
from jax.experimental.pallas import tpu as pltpu
from jax.experimental.pallas import tpu_sc as plsc

N_NODES = 10000
N_EDGES = 320000
D_FEAT = 128

NC = 2
NS = 16
NW = NC * NS
E_TILE = N_EDGES // NW
CHUNK = 80
CHUNKS = E_TILE // CHUNK
SUB = CHUNK // 16
ROWS_TILE = N_NODES // NS

_mesh = plsc.VectorSubcoreMesh(
    core_axis_name="c", subcore_axis_name="s", num_cores=NC, num_subcores=NS
)


@functools.partial(
    pl.kernel,
    out_type=jax.ShapeDtypeStruct((NC, N_NODES, D_FEAT), jnp.float32),
    mesh=_mesh,
    compiler_params=pltpu.CompilerParams(use_tc_tiling_on_sc=False),
    scratch_types=[
        pltpu.VMEM((CHUNKS, CHUNK), jnp.int32),
        pltpu.VMEM((CHUNKS, CHUNK), jnp.int32),
        pltpu.VMEM((E_TILE,), jnp.float32),
        pltpu.VMEM((CHUNK, D_FEAT), jnp.float32),
        pltpu.VMEM((CHUNK, D_FEAT), jnp.float32),
        pltpu.VMEM_SHARED((N_NODES, D_FEAT), jnp.float32),
        pltpu.SemaphoreType.DMA,
        pltpu.SemaphoreType.DMA,
        pltpu.SemaphoreType.DMA,
        pltpu.SemaphoreType.DMA,
    ],
)
def _sc_scatter(x_hbm, src_hbm, tgt_hbm, val_hbm, zero_hbm, out_hbm,
                src_v, tgt_v, val_v, rows_a, rows_b, acc_sh,
                sga, sgb, ssa, ssb):
    cid = lax.axis_index("c")
    sid = lax.axis_index("s")
    wid = sid * NC + cid

    pltpu.sync_copy(src_hbm.at[wid], src_v)
    pltpu.sync_copy(tgt_hbm.at[wid], tgt_v)
    pltpu.sync_copy(val_hbm.at[wid], val_v)

    pltpu.sync_copy(
        zero_hbm, acc_sh.at[pl.ds(sid * ROWS_TILE, ROWS_TILE)]
    )
    plsc.subcore_barrier()

    def _g_start(j, buf, sem):
        pltpu.async_copy(x_hbm.at[src_v.at[j]], buf, sem)

    def _g_wait(j, buf, sem):
        pltpu.make_async_copy(x_hbm.at[src_v.at[j]], buf, sem).wait()

    def _s_start(j, buf, sem):
        pltpu.async_copy(buf, acc_sh.at[tgt_v.at[j]], sem, add=True)

    def _s_wait(j, buf, sem):
        pltpu.make_async_copy(buf, acc_sh.at[tgt_v.at[j]], sem).wait()

    def _scale(j, buf):
        def _group(s, c2):
            off = pl.multiple_of(j * CHUNK + s * 16, 16)
            val16 = val_v[pl.ds(off, 16)]
            for e16 in range(16):
                sv = jnp.take_along_axis(
                    val16, jnp.full((16,), e16, jnp.int32), axis=0
                )
                e = s * 16 + e16
                for k in range(D_FEAT // 16):
                    buf[e, pl.ds(k * 16, 16)] = (
                        buf[e, pl.ds(k * 16, 16)] * sv
                    )
            return c2

        lax.fori_loop(0, SUB, _group, 0)

    _g_start(0, rows_a, sga)

    def _pair(i, carry):
        a = 2 * i
        b = a + 1
        _g_wait(a, rows_a, sga)

        @pl.when(i > 0)
        def _():
            _s_wait(b - 2, rows_b, ssb)

        _g_start(b, rows_b, sgb)
        _scale(a, rows_a)
        _s_start(a, rows_a, ssa)
        _g_wait(b, rows_b, sgb)
        _s_wait(a, rows_a, ssa)
        _g_start(a + 2, rows_a, sga)
        _scale(b, rows_b)
        _s_start(b, rows_b, ssb)
        return carry

    lax.fori_loop(0, (CHUNKS - 1) // 2, _pair, 0)

    last = CHUNKS - 1
    _g_wait(last, rows_a, sga)
    _s_wait(last - 1, rows_b, ssb)
    _scale(last, rows_a)
    _s_start(last, rows_a, ssa)
    _s_wait(last, rows_a, ssa)
    plsc.subcore_barrier()

    pltpu.sync_copy(
        acc_sh.at[pl.ds(sid * ROWS_TILE, ROWS_TILE)],
        out_hbm.at[cid, pl.ds(sid * ROWS_TILE, ROWS_TILE)],
    )


def _combine_body(a_ref, b_ref, o_ref):
    o_ref[...] = a_ref[...] + b_ref[...]


_combine = pl.pallas_call(
    _combine_body,
    out_shape=jax.ShapeDtypeStruct((N_NODES, D_FEAT), jnp.float32),
    grid=(5,),
    in_specs=[
        pl.BlockSpec((2000, D_FEAT), lambda i: (i, 0)),
        pl.BlockSpec((2000, D_FEAT), lambda i: (i, 0)),
    ],
    out_specs=pl.BlockSpec((2000, D_FEAT), lambda i: (i, 0)),
)


@jax.jit
def kernel(x_source, edge_index, values):
    src = edge_index[1].reshape(NW, CHUNKS, CHUNK)
    tgt = edge_index[0].reshape(NW, CHUNKS, CHUNK)
    val = values.reshape(NW, E_TILE)
    zero = jnp.zeros((ROWS_TILE, D_FEAT), jnp.float32)
    partial = _sc_scatter(x_source, src, tgt, val, zero)
    return _combine(partial[0], partial[1])
